# baseline (speedup 1.0000x reference)
import functools

import jax
import jax.numpy as jnp
from jax import lax
from jax.experimental import pallas as pl
from jax.experimental.pallas import tpu as pltpu
from jax.experimental.pallas import tpu_sc as plsc

SPLIT_INTO = 7
PERM = (2, 4, 3, 6, 5, 0, 1)


def kernel(x):
    b, t, f = x.shape
    chunk = t // SPLIT_INTO
    row_words = chunk * f
    rows = b * SPLIT_INTO

    info = plsc.get_sparse_core_info()
    nc, ns = info.num_cores, info.num_subcores
    nw = nc * ns
    batches_per_w = b // nw

    xr = x.reshape(rows, row_words)
    mesh = plsc.VectorSubcoreMesh(core_axis_name="c", subcore_axis_name="s")

    @functools.partial(
        pl.kernel,
        mesh=mesh,
        out_type=jax.ShapeDtypeStruct((rows, row_words), jnp.float32),
        scratch_types=[pltpu.SemaphoreType.DMA],
    )
    def reorder(x_hbm, out_hbm, sem):
        wid = lax.axis_index("s") * nc + lax.axis_index("c")
        b0 = wid * batches_per_w
        copies = []
        for bi in range(batches_per_w):
            for c in range(SPLIT_INTO):
                base = (b0 + bi) * SPLIT_INTO
                copies.append(
                    pltpu.async_copy(
                        x_hbm.at[base + PERM[c]], out_hbm.at[base + c], sem
                    )
                )
        for cp in copies:
            cp.wait()

    out = reorder(xr)
    return out.reshape(x.shape)


# speedup vs baseline: 1.5977x; 1.5977x over previous
import functools

import jax
import jax.numpy as jnp
from jax import lax
from jax.experimental import pallas as pl
from jax.experimental.pallas import tpu as pltpu
from jax.experimental.pallas import tpu_sc as plsc

SPLIT_INTO = 7
PERM = (2, 4, 3, 6, 5, 0, 1)
NBUF = 4
AHEAD = 2


def kernel(x):
    b, t, f = x.shape
    chunk = t // SPLIT_INTO
    row_words = chunk * f
    rows = b * SPLIT_INTO

    info = plsc.get_sparse_core_info()
    nc, ns = info.num_cores, info.num_subcores
    nw = nc * ns
    batches_per_w = b // nw
    n = batches_per_w * SPLIT_INTO

    xr = x.reshape(rows, row_words)
    mesh = plsc.VectorSubcoreMesh(core_axis_name="c", subcore_axis_name="s")

    @functools.partial(
        pl.kernel,
        mesh=mesh,
        out_type=jax.ShapeDtypeStruct((rows, row_words), jnp.float32),
        scratch_types=[
            pltpu.VMEM((NBUF, row_words), jnp.float32),
            pltpu.SemaphoreType.DMA((NBUF,)),
            pltpu.SemaphoreType.DMA((NBUF,)),
        ],
    )
    def reorder(x_hbm, out_hbm, buf, sem_in, sem_out):
        wid = lax.axis_index("s") * nc + lax.axis_index("c")
        base = wid * batches_per_w * SPLIT_INTO

        def src_row(j):
            return base + (j // SPLIT_INTO) * SPLIT_INTO + PERM[j % SPLIT_INTO]

        def start_in(j):
            return pltpu.async_copy(
                x_hbm.at[src_row(j)], buf.at[j % NBUF], sem_in.at[j % NBUF]
            )

        def start_out(j):
            return pltpu.async_copy(
                buf.at[j % NBUF], out_hbm.at[base + j], sem_out.at[j % NBUF]
            )

        ins = {j: start_in(j) for j in range(AHEAD)}
        outs = {}
        for j in range(n):
            k = j + AHEAD
            if k < n:
                if k >= NBUF:
                    outs[k - NBUF].wait()
                ins[k] = start_in(k)
            ins[j].wait()
            outs[j] = start_out(j)
        for j in range(max(0, n - NBUF), n):
            outs[j].wait()

    out = reorder(xr)
    return out.reshape(x.shape)


# speedup vs baseline: 2.6536x; 1.6609x over previous
import functools

import jax
import jax.numpy as jnp
from jax import lax
from jax.experimental import pallas as pl
from jax.experimental.pallas import tpu as pltpu
from jax.experimental.pallas import tpu_sc as plsc

SPLIT_INTO = 7
PERM = (2, 4, 3, 6, 5, 0, 1)
NBUF = 4
AHEAD = 2


def kernel(x):
    b, t, f = x.shape
    chunk = t // SPLIT_INTO

    info = plsc.get_sparse_core_info()
    nc, ns = info.num_cores, info.num_subcores
    nw = nc * ns
    batches_per_w = b // nw
    n = batches_per_w * SPLIT_INTO

    mesh = plsc.VectorSubcoreMesh(core_axis_name="c", subcore_axis_name="s")

    @functools.partial(
        pl.kernel,
        mesh=mesh,
        out_type=jax.ShapeDtypeStruct((b, t, f), jnp.float32),
        compiler_params=pltpu.CompilerParams(use_tc_tiling_on_sc=False),
        scratch_types=[
            pltpu.VMEM((NBUF, chunk, f), jnp.float32),
            pltpu.SemaphoreType.DMA((NBUF,)),
            pltpu.SemaphoreType.DMA((NBUF,)),
        ],
    )
    def reorder(x_hbm, out_hbm, buf, sem_in, sem_out):
        wid = lax.axis_index("s") * nc + lax.axis_index("c")
        b0 = wid * batches_per_w

        def start_in(j):
            bi, c = divmod(j, SPLIT_INTO)
            return pltpu.async_copy(
                x_hbm.at[b0 + bi, pl.ds(PERM[c] * chunk, chunk)],
                buf.at[j % NBUF],
                sem_in.at[j % NBUF],
            )

        def start_out(j):
            bi, c = divmod(j, SPLIT_INTO)
            return pltpu.async_copy(
                buf.at[j % NBUF],
                out_hbm.at[b0 + bi, pl.ds(c * chunk, chunk)],
                sem_out.at[j % NBUF],
            )

        ins = {j: start_in(j) for j in range(AHEAD)}
        outs = {}
        for j in range(n):
            k = j + AHEAD
            if k < n:
                if k >= NBUF:
                    outs[k - NBUF].wait()
                ins[k] = start_in(k)
            ins[j].wait()
            outs[j] = start_out(j)
        for j in range(max(0, n - NBUF), n):
            outs[j].wait()

    return reorder(x)
